# Initial kernel scaffold; baseline (speedup 1.0000x reference)
import jax
import jax.numpy as jnp
from jax.experimental import pallas as pl


def kernel(input_text, input_img, input_compress, base_text_features, base_img_features, Wu, bu, Wl_img, bl_img, Wr_img, Wl_txt, bl_txt, Wr_txt):
    raise NotImplementedError("write your pallas kernel here")


# speedup vs baseline: 21.0193x; 21.0193x over previous
import functools

import jax
import jax.numpy as jnp
from jax.experimental import pallas as pl


def _fused_body(uf_ref, text_ref, img_ref, Wu_ref, bu_ref, Wl_img_ref,
                bl_img_ref, Wl_txt_ref, bl_txt_ref, Wr_sum_ref, out_ref,
                *, inv_m):
    agg_t = jnp.sum(text_ref[...], axis=1) * inv_m
    agg_i = jnp.sum(img_ref[...], axis=1) * inv_m
    user_x = (
        jnp.dot(uf_ref[...], Wu_ref[...], preferred_element_type=jnp.float32)
        + bu_ref[...]
    )
    acc = jnp.dot(agg_i, Wl_img_ref[...], preferred_element_type=jnp.float32)
    acc += jnp.dot(agg_t, Wl_txt_ref[...], preferred_element_type=jnp.float32)
    acc += jnp.dot(user_x, Wr_sum_ref[...], preferred_element_type=jnp.float32)
    acc += bl_img_ref[...] + bl_txt_ref[...]
    out_ref[...] = jnp.maximum(acc, 0.0)


def kernel(input_text, input_img, input_compress, base_text_features,
           base_img_features, Wu, bu, Wl_img, bl_img, Wr_img,
           Wl_txt, bl_txt, Wr_txt):
    b, m, h = base_text_features.shape
    feat = Wu.shape[0]

    user_feat = jnp.concatenate(
        [input_text[:, 0, :], input_img[:, 0, :], input_compress], axis=1)

    bu2 = bu.reshape(1, h)
    bl_img2 = bl_img.reshape(1, h)
    bl_txt2 = bl_txt.reshape(1, h)
    Wr_sum = Wr_img + Wr_txt

    bb = 16
    while b % bb:
        bb //= 2
    grid = (b // bb,)

    body = functools.partial(_fused_body, inv_m=1.0 / m)

    full2 = lambda i: (0, 0)
    return pl.pallas_call(
        body,
        grid=grid,
        in_specs=[
            pl.BlockSpec((bb, feat), lambda i: (i, 0)),
            pl.BlockSpec((bb, m, h), lambda i: (i, 0, 0)),
            pl.BlockSpec((bb, m, h), lambda i: (i, 0, 0)),
            pl.BlockSpec((feat, h), full2),
            pl.BlockSpec((1, h), full2),
            pl.BlockSpec((h, h), full2),
            pl.BlockSpec((1, h), full2),
            pl.BlockSpec((h, h), full2),
            pl.BlockSpec((1, h), full2),
            pl.BlockSpec((h, h), full2),
        ],
        out_specs=pl.BlockSpec((bb, h), lambda i: (i, 0)),
        out_shape=jax.ShapeDtypeStruct((b, h), jnp.float32),
    )(user_feat, base_text_features, base_img_features, Wu, bu2,
      Wl_img, bl_img2, Wl_txt, bl_txt2, Wr_sum)
